# Initial kernel scaffold; baseline (speedup 1.0000x reference)
import jax
import jax.numpy as jnp
from jax.experimental import pallas as pl


def kernel(x, token_table, pos_table):
    raise NotImplementedError("write your pallas kernel here")


# speedup vs baseline: 3.4659x; 3.4659x over previous
import functools

import jax
import jax.numpy as jnp
from jax import lax
from jax.experimental import pallas as pl
from jax.experimental.pallas import tpu as pltpu
from jax.experimental.pallas import tpu_sc as plsc

_NC = 2
_NS = 16
_NW = _NC * _NS
_LANES = 16


def kernel(x, token_table, pos_table):
    B, T = x.shape
    V, E = token_table.shape
    ncol = E // _LANES

    rows_per_w = (B // _NW) * T
    chunk_rows = 2
    ch = chunk_rows * T
    n_chunks = (B // _NW) // chunk_rows

    x_flat = x.reshape(-1).astype(jnp.int32)

    mesh = plsc.VectorSubcoreMesh(core_axis_name="c", subcore_axis_name="s")

    @functools.partial(
        pl.kernel,
        mesh=mesh,
        compiler_params=pltpu.CompilerParams(use_tc_tiling_on_sc=False),
        out_type=jax.ShapeDtypeStruct((B * T, E), jnp.float32),
        scratch_types=[
            pltpu.VMEM((T, E), jnp.float32),
            pltpu.VMEM((ch,), jnp.int32),
            pltpu.VMEM((ch, E), jnp.float32),
            pltpu.SemaphoreType.DMA,
        ],
    )
    def emb_kernel(x_hbm, tok_hbm, pos_hbm, out_hbm, pos_v, idx_v, rows_v, sem):
        wid = lax.axis_index("s") * _NC + lax.axis_index("c")
        base = wid * rows_per_w
        pltpu.sync_copy(pos_hbm, pos_v)

        def chunk_body(i, carry):
            off = base + i * ch
            pltpu.sync_copy(x_hbm.at[pl.ds(off, ch)], idx_v)
            pltpu.async_copy(tok_hbm.at[idx_v], rows_v, sem).wait()

            def add_pos(p, c2):
                pv = [pos_v[p, pl.ds(c * _LANES, _LANES)] for c in range(ncol)]
                for r in range(chunk_rows):
                    row = r * T + p
                    for c in range(ncol):
                        sl = pl.ds(c * _LANES, _LANES)
                        rows_v[row, sl] = rows_v[row, sl] + pv[c]
                return c2

            lax.fori_loop(0, T, add_pos, 0)
            pltpu.sync_copy(rows_v, out_hbm.at[pl.ds(off, ch)])
            return carry

        lax.fori_loop(0, n_chunks, chunk_body, 0)

    out = emb_kernel(x_flat, token_table, pos_table)
    return out.reshape(B, T, E)


# speedup vs baseline: 4.1950x; 1.2104x over previous
import functools

import jax
import jax.numpy as jnp
from jax import lax
from jax.experimental import pallas as pl
from jax.experimental.pallas import tpu as pltpu
from jax.experimental.pallas import tpu_sc as plsc

_NC = 2
_NS = 16
_NW = _NC * _NS
_LANES = 16


def kernel(x, token_table, pos_table):
    B, T = x.shape
    V, E = token_table.shape
    ncol = E // _LANES

    rows_per_w = (B // _NW) * T
    chunk_rows = 2
    ch = chunk_rows * T
    n_chunks = (B // _NW) // chunk_rows
    half = n_chunks // 2

    x_flat = x.reshape(-1).astype(jnp.int32)

    mesh = plsc.VectorSubcoreMesh(core_axis_name="c", subcore_axis_name="s")

    @functools.partial(
        pl.kernel,
        mesh=mesh,
        compiler_params=pltpu.CompilerParams(use_tc_tiling_on_sc=False),
        out_type=jax.ShapeDtypeStruct((B * T, E), jnp.float32),
        scratch_types=[
            pltpu.VMEM((T, E), jnp.float32),
            pltpu.VMEM((rows_per_w,), jnp.int32),
            pltpu.VMEM((2, ch, E), jnp.float32),
            pltpu.SemaphoreType.DMA,
            pltpu.SemaphoreType.DMA,
            pltpu.SemaphoreType.DMA,
            pltpu.SemaphoreType.DMA,
        ],
    )
    def emb_kernel(x_hbm, tok_hbm, pos_hbm, out_hbm,
                   pos_v, idx_v, rows_v, g0, g1, s0, s1):
        gsem = (g0, g1)
        ssem = (s0, s1)
        wid = lax.axis_index("s") * _NC + lax.axis_index("c")
        base = wid * rows_per_w

        pltpu.sync_copy(x_hbm.at[pl.ds(base, rows_per_w)], idx_v)
        pltpu.sync_copy(pos_hbm, pos_v)

        def gather_copy(i, b):
            return pltpu.make_async_copy(
                tok_hbm.at[idx_v.at[pl.ds(i * ch, ch)]], rows_v.at[b], gsem[b])

        def store_copy(i, b):
            return pltpu.make_async_copy(
                rows_v.at[b], out_hbm.at[pl.ds(base + i * ch, ch)], ssem[b])

        def add_pos(b):
            def body(p, carry):
                pv = [pos_v[p, pl.ds(c * _LANES, _LANES)] for c in range(ncol)]
                for r in range(chunk_rows):
                    row = r * T + p
                    for c in range(ncol):
                        sl = pl.ds(c * _LANES, _LANES)
                        rows_v[b, row, sl] = rows_v[b, row, sl] + pv[c]
                return carry
            lax.fori_loop(0, T, body, 0)

        gather_copy(0, 0).start()

        def g_body(g, carry):
            i0 = 2 * g
            gather_copy(i0, 0).wait()

            @pl.when(g >= 1)
            def _():
                store_copy(i0 - 1, 1).wait()
            gather_copy(i0 + 1, 1).start()
            add_pos(0)
            store_copy(i0, 0).start()
            gather_copy(i0 + 1, 1).wait()
            store_copy(i0, 0).wait()

            @pl.when(g < half - 1)
            def _():
                gather_copy(i0 + 2, 0).start()
            add_pos(1)
            store_copy(i0 + 1, 1).start()
            return carry

        lax.fori_loop(0, half, g_body, 0)
        store_copy(n_chunks - 1, 1).wait()

    out = emb_kernel(x_flat, token_table, pos_table)
    return out.reshape(B, T, E)
